# Initial kernel scaffold; baseline (speedup 1.0000x reference)
import jax
import jax.numpy as jnp
from jax.experimental import pallas as pl


def kernel(nodes_feat, edge_index, edges_feat, nodes_num_norm_sqrt, edges_num_norm_sqrt, W_emb, b_emb, W1, b1, gamma1, beta1, Wo, bo, gammao, betao):
    raise NotImplementedError("write your pallas kernel here")


# speedup vs baseline: 1.6437x; 1.6437x over previous
import functools

import jax
import jax.numpy as jnp
from jax import lax
from jax.experimental import pallas as pl
from jax.experimental.pallas import tpu as pltpu
from jax.experimental.pallas import tpu_sc as plsc

N = 50000
NPAD = 50048
E = 800000
EPAD = 819200
F = 146
FPAD = 160
C = 16
NCH = 10
DIN = 64
BN = 2000
GRID = N // BN
BNB = 1000
GRIDB = N // BNB
NC = 2
NS = 16
ROWS_PER_TILE_DEG = 400
ROWS_PER_TILE_AGG = 200

_f32 = jnp.float32


def _mesh():
    return plsc.VectorSubcoreMesh(
        core_axis_name="c", subcore_axis_name="s", num_cores=NC, num_subcores=NS
    )


def _zero_parts(total, buf):
    parts = []
    off = 0
    while off < total:
        sz = min(buf, total - off)
        parts.append((off, sz))
        off += sz
    return parts


def _sc_degrees(src2d, dst2d, zeros1_h):
    EPT = EPAD // NS

    @functools.partial(
        pl.kernel,
        out_type=[jax.ShapeDtypeStruct((NS, N), _f32),
                  jax.ShapeDtypeStruct((NS, N), _f32)],
        mesh=_mesh(),
        compiler_params=pltpu.CompilerParams(
            use_tc_tiling_on_sc=False, needs_layout_passes=False),
        scratch_types=[
            pltpu.VMEM((EPT,), jnp.int32),
            pltpu.VMEM((NPAD,), _f32),
        ],
    )
    def deg_kernel(src_h, dst_h, zeros_hbm, out_o, out_i, idx_v, table):
        cid = lax.axis_index("c")
        sid = lax.axis_index("s")
        pltpu.sync_copy(zeros_hbm, table)

        for k in range(NS):
            @pl.when((cid == 0) & (sid == k))
            def _(k=k):
                pltpu.sync_copy(src_h.at[pl.ds(k * EPT, EPT)], idx_v)

            @pl.when((cid == 1) & (sid == k))
            def _(k=k):
                pltpu.sync_copy(dst_h.at[pl.ds(k * EPT, EPT)], idx_v)

        ones16 = jnp.ones((16,), _f32)

        def body(j, carry):
            idx16 = idx_v[pl.ds(j * 16, 16)]
            plsc.addupdate_scatter(table, [idx16], ones16)
            return carry

        lax.fori_loop(0, EPT // 16, body, 0)

        for k in range(NS):
            @pl.when((cid == 0) & (sid == k))
            def _(k=k):
                pltpu.sync_copy(table.at[pl.ds(0, N)], out_o.at[k])

            @pl.when((cid == 1) & (sid == k))
            def _(k=k):
                pltpu.sync_copy(table.at[pl.ds(0, N)], out_i.at[k])

    return deg_kernel(src2d, dst2d, zeros1_h)


def _sc_aggregate(hs_list, src2d, dst2d, zeros32_h):
    out_types = [jax.ShapeDtypeStruct((NC, N, C), _f32) for _ in range(NCH)]

    @functools.partial(
        pl.kernel,
        out_type=out_types,
        mesh=_mesh(),
        compiler_params=pltpu.CompilerParams(use_tc_tiling_on_sc=False),
        scratch_types=[
            pltpu.VMEM((ROWS_PER_TILE_AGG, 128), jnp.int32),
            pltpu.VMEM((ROWS_PER_TILE_AGG, 128), jnp.int32),
            pltpu.VMEM((128, C), _f32),
            pltpu.VMEM((1024, C), _f32),
            pltpu.VMEM_SHARED((NPAD, C), _f32),
            pltpu.SemaphoreType.DMA,
        ],
    )
    def agg_kernel(*args):
        hs_refs = args[:NCH]
        src_h, dst_h, zeros_hbm = args[NCH:NCH + 3]
        outs = args[NCH + 3:2 * NCH + 3]
        src_l, dst_l, rows, zb, table, sem = args[2 * NCH + 3:]
        cid = lax.axis_index("c")
        sid = lax.axis_index("s")
        pltpu.sync_copy(zeros_hbm, zb)
        rbase = cid * (NS * ROWS_PER_TILE_AGG) + sid * ROWS_PER_TILE_AGG
        pltpu.sync_copy(src_h.at[pl.ds(rbase, ROWS_PER_TILE_AGG)], src_l)
        pltpu.sync_copy(dst_h.at[pl.ds(rbase, ROWS_PER_TILE_AGG)], dst_l)
        base = sid * 3125
        for ch in range(NCH):
            for off, sz in _zero_parts(3125, 1024):
                pltpu.sync_copy(zb.at[pl.ds(0, sz)], table.at[pl.ds(base + off, sz)])
            plsc.subcore_barrier()

            def body(j, carry):
                pltpu.async_copy(hs_refs[ch].at[src_l.at[j]], rows, sem).wait()
                pltpu.sync_copy(rows, table.at[dst_l.at[j]], add=True)
                return carry

            lax.fori_loop(0, ROWS_PER_TILE_AGG, body, 0)
            plsc.subcore_barrier()
            pltpu.sync_copy(
                table.at[pl.ds(base, 3125)], outs[ch].at[cid, pl.ds(base, 3125)]
            )
            plsc.subcore_barrier()

    return agg_kernel(*hs_list, src2d, dst2d, zeros32_h)


def _tc_embed(nodes_feat, W_emb, b_emb2, dgo_p, dgi_p):
    def body(nf, we, be, dgo_r, dgi_r, h_o, ns_o, nd_o, *hs_refs):
        x = jnp.dot(nf[...], we[...], preferred_element_type=_f32,
                    precision=lax.Precision.HIGHEST) + be[...]
        dgo = jnp.sum(dgo_r[...], axis=1, keepdims=True)
        dgi = jnp.sum(dgi_r[...], axis=1, keepdims=True)
        ns = jnp.where(dgo > 0, lax.rsqrt(jnp.maximum(dgo, 1.0)), 0.0)
        nd = jnp.where(dgi > 0, lax.rsqrt(jnp.maximum(dgi, 1.0)), 0.0)
        h_o[...] = x
        ns_o[...] = ns
        nd_o[...] = nd
        hp = x * ns
        hpad = jnp.concatenate([hp, jnp.zeros((BN, FPAD - F), _f32)], axis=1)
        for ch in range(NCH):
            hs_refs[ch][...] = hpad[:, ch * C:(ch + 1) * C]

    out_shapes = (
        [jax.ShapeDtypeStruct((N, F), _f32),
         jax.ShapeDtypeStruct((N, 1), _f32),
         jax.ShapeDtypeStruct((N, 1), _f32)]
        + [jax.ShapeDtypeStruct((NPAD, C), _f32) for _ in range(NCH)]
    )
    out_specs = (
        [pl.BlockSpec((BN, F), lambda i: (i, 0)),
         pl.BlockSpec((BN, 1), lambda i: (i, 0)),
         pl.BlockSpec((BN, 1), lambda i: (i, 0))]
        + [pl.BlockSpec((BN, C), lambda i: (i, 0)) for _ in range(NCH)]
    )
    return pl.pallas_call(
        body,
        grid=(GRID,),
        compiler_params=pltpu.CompilerParams(vmem_limit_bytes=61440000),
        in_specs=[
            pl.BlockSpec((BN, DIN), lambda i: (i, 0)),
            pl.BlockSpec((DIN, F), lambda i: (0, 0)),
            pl.BlockSpec((1, F), lambda i: (0, 0)),
            pl.BlockSpec((BN, NS), lambda i: (i, 0)),
            pl.BlockSpec((BN, NS), lambda i: (i, 0)),
        ],
        out_specs=out_specs,
        out_shape=out_shapes,
    )(nodes_feat, W_emb, b_emb2, dgo_p, dgi_p)


def _tc_linear_stats(agg_list, nd, Wp, b2, snorm):
    def body(*args):
        aggs = args[:NCH]
        nd_r, w_r, b_r, sn_r, z_o, s_o, q_o = args[NCH:]
        i = pl.program_id(0)
        ndv = nd_r[...]
        acc = jnp.zeros((BNB, F), _f32)
        for ch in range(NCH):
            a = (aggs[ch][0] + aggs[ch][1]) * ndv
            acc = acc + jnp.dot(a, w_r[ch * C:(ch + 1) * C, :],
                                preferred_element_type=_f32, precision=lax.Precision.HIGHEST)
        z = (acc + b_r[...]) * sn_r[...]
        z_o[...] = z
        cs = jnp.sum(z, axis=0, keepdims=True)
        cq = jnp.sum(z * z, axis=0, keepdims=True)

        @pl.when(i == 0)
        def _():
            s_o[...] = cs
            q_o[...] = cq

        @pl.when(i > 0)
        def _():
            s_o[...] = s_o[...] + cs
            q_o[...] = q_o[...] + cq

    return pl.pallas_call(
        body,
        grid=(GRIDB,),
        compiler_params=pltpu.CompilerParams(vmem_limit_bytes=61440000),
        in_specs=(
            [pl.BlockSpec((NC, BNB, C), lambda i: (0, i, 0)) for _ in range(NCH)]
            + [pl.BlockSpec((BNB, 1), lambda i: (i, 0)),
               pl.BlockSpec((FPAD, F), lambda i: (0, 0)),
               pl.BlockSpec((1, F), lambda i: (0, 0)),
               pl.BlockSpec((BNB, 1), lambda i: (i, 0))]
        ),
        out_specs=[
            pl.BlockSpec((BNB, F), lambda i: (i, 0)),
            pl.BlockSpec((1, F), lambda i: (0, 0)),
            pl.BlockSpec((1, F), lambda i: (0, 0)),
        ],
        out_shape=[
            jax.ShapeDtypeStruct((N, F), _f32),
            jax.ShapeDtypeStruct((1, F), _f32),
            jax.ShapeDtypeStruct((1, F), _f32),
        ],
    )(*agg_list, nd, Wp, b2, snorm)


def _tc_norm_residual(Z, h, s1, q1, gamma2, beta2, ns):
    def body(z_r, h_r, s_r, q_r, g_r, be_r, ns_r, h2_o, *hs_refs):
        mean = s_r[...] * (1.0 / N)
        var = q_r[...] * (1.0 / N) - mean * mean
        inv = lax.rsqrt(var + 1e-5)
        out = h_r[...] + jnp.maximum((z_r[...] - mean) * inv * g_r[...] + be_r[...], 0.0)
        h2_o[...] = out
        hp = out * ns_r[...]
        hpad = jnp.concatenate([hp, jnp.zeros((BN, FPAD - F), _f32)], axis=1)
        for ch in range(NCH):
            hs_refs[ch][...] = hpad[:, ch * C:(ch + 1) * C]

    out_shapes = (
        [jax.ShapeDtypeStruct((N, F), _f32)]
        + [jax.ShapeDtypeStruct((NPAD, C), _f32) for _ in range(NCH)]
    )
    out_specs = (
        [pl.BlockSpec((BN, F), lambda i: (i, 0))]
        + [pl.BlockSpec((BN, C), lambda i: (i, 0)) for _ in range(NCH)]
    )
    return pl.pallas_call(
        body,
        grid=(GRID,),
        compiler_params=pltpu.CompilerParams(vmem_limit_bytes=61440000),
        in_specs=[
            pl.BlockSpec((BN, F), lambda i: (i, 0)),
            pl.BlockSpec((BN, F), lambda i: (i, 0)),
            pl.BlockSpec((1, F), lambda i: (0, 0)),
            pl.BlockSpec((1, F), lambda i: (0, 0)),
            pl.BlockSpec((1, F), lambda i: (0, 0)),
            pl.BlockSpec((1, F), lambda i: (0, 0)),
            pl.BlockSpec((BN, 1), lambda i: (i, 0)),
        ],
        out_specs=out_specs,
        out_shape=out_shapes,
    )(Z, h, s1, q1, gamma2, beta2, ns)


def _tc_norm_readout(Z, h, s1, q1, gamma2, beta2):
    def body(z_r, h_r, s_r, q_r, g_r, be_r, hg_o):
        i = pl.program_id(0)
        mean = s_r[...] * (1.0 / N)
        var = q_r[...] * (1.0 / N) - mean * mean
        inv = lax.rsqrt(var + 1e-5)
        out = h_r[...] + jnp.maximum((z_r[...] - mean) * inv * g_r[...] + be_r[...], 0.0)
        cs = jnp.sum(out, axis=0, keepdims=True) * (1.0 / N)

        @pl.when(i == 0)
        def _():
            hg_o[...] = cs

        @pl.when(i > 0)
        def _():
            hg_o[...] = hg_o[...] + cs

    return pl.pallas_call(
        body,
        grid=(GRID,),
        compiler_params=pltpu.CompilerParams(vmem_limit_bytes=61440000),
        in_specs=[
            pl.BlockSpec((BN, F), lambda i: (i, 0)),
            pl.BlockSpec((BN, F), lambda i: (i, 0)),
            pl.BlockSpec((1, F), lambda i: (0, 0)),
            pl.BlockSpec((1, F), lambda i: (0, 0)),
            pl.BlockSpec((1, F), lambda i: (0, 0)),
            pl.BlockSpec((1, F), lambda i: (0, 0)),
        ],
        out_specs=pl.BlockSpec((1, F), lambda i: (0, 0)),
        out_shape=jax.ShapeDtypeStruct((1, F), _f32),
    )(Z, h, s1, q1, gamma2, beta2)


def kernel(nodes_feat, edge_index, edges_feat, nodes_num_norm_sqrt,
           edges_num_norm_sqrt, W_emb, b_emb, W1, b1, gamma1, beta1,
           Wo, bo, gammao, betao):
    del edges_feat, edges_num_norm_sqrt
    src = edge_index[0].astype(jnp.int32)
    dst = edge_index[1].astype(jnp.int32)
    fill = jnp.full((EPAD - E,), N, dtype=jnp.int32)
    src_flat = jnp.concatenate([src, fill])
    dst_flat = jnp.concatenate([dst, fill])
    src2d = src_flat.reshape(EPAD // 128, 128)
    dst2d = dst_flat.reshape(EPAD // 128, 128)

    zeros32_h = jnp.zeros((1024, C), _f32)
    zerosN_h = jnp.zeros((NPAD,), _f32)

    b_emb2 = b_emb.reshape(1, F)
    b12 = b1.reshape(1, F)
    bo2 = bo.reshape(1, F)
    gamma12 = gamma1.reshape(1, F)
    beta12 = beta1.reshape(1, F)
    gammao2 = gammao.reshape(1, F)
    betao2 = betao.reshape(1, F)
    W1p = jnp.pad(W1, ((0, FPAD - F), (0, 0)))
    Wop = jnp.pad(Wo, ((0, FPAD - F), (0, 0)))
    snorm = nodes_num_norm_sqrt

    dgo_p, dgi_p = _sc_degrees(src_flat, dst_flat, zerosN_h)
    dgo_p = dgo_p.T
    dgi_p = dgi_p.T

    h, ns, nd, *hs_list = _tc_embed(nodes_feat, W_emb, b_emb2, dgo_p, dgi_p)

    agg1 = _sc_aggregate(hs_list, src2d, dst2d, zeros32_h)
    Z1, s1, q1 = _tc_linear_stats(agg1, nd, W1p, b12, snorm)
    h2, *hs2_list = _tc_norm_residual(Z1, h, s1, q1, gamma12, beta12, ns)

    agg2 = _sc_aggregate(hs2_list, src2d, dst2d, zeros32_h)
    Z2, s2, q2 = _tc_linear_stats(agg2, nd, Wop, bo2, snorm)
    hg = _tc_norm_readout(Z2, h2, s2, q2, gammao2, betao2)
    return hg


# speedup vs baseline: 2.0173x; 1.2273x over previous
import functools

import jax
import jax.numpy as jnp
from jax import lax
from jax.experimental import pallas as pl
from jax.experimental.pallas import tpu as pltpu
from jax.experimental.pallas import tpu_sc as plsc

N = 50000
NPAD = 50048
E = 800000
EPAD = 819200
F = 146
FPAD = 160
C = 16
NCH = 10
DIN = 64
BN = 2000
GRID = N // BN
BNB = 1000
GRIDB = N // BNB
NC = 2
NS = 16
ROWS_PER_TILE_DEG = 400
ROWS_PER_TILE_AGG = 200

_f32 = jnp.float32


def _mesh():
    return plsc.VectorSubcoreMesh(
        core_axis_name="c", subcore_axis_name="s", num_cores=NC, num_subcores=NS
    )


def _zero_parts(total, buf):
    parts = []
    off = 0
    while off < total:
        sz = min(buf, total - off)
        parts.append((off, sz))
        off += sz
    return parts


def _sc_degrees(src2d, dst2d, zeros1_h):
    EPT = EPAD // NS

    @functools.partial(
        pl.kernel,
        out_type=[jax.ShapeDtypeStruct((NS, N), _f32),
                  jax.ShapeDtypeStruct((NS, N), _f32)],
        mesh=_mesh(),
        compiler_params=pltpu.CompilerParams(
            use_tc_tiling_on_sc=False, needs_layout_passes=False),
        scratch_types=[
            pltpu.VMEM((EPT,), jnp.int32),
            pltpu.VMEM((NPAD,), _f32),
        ],
    )
    def deg_kernel(src_h, dst_h, zeros_hbm, out_o, out_i, idx_v, table):
        cid = lax.axis_index("c")
        sid = lax.axis_index("s")
        pltpu.sync_copy(zeros_hbm, table)

        for k in range(NS):
            @pl.when((cid == 0) & (sid == k))
            def _(k=k):
                pltpu.sync_copy(src_h.at[pl.ds(k * EPT, EPT)], idx_v)

            @pl.when((cid == 1) & (sid == k))
            def _(k=k):
                pltpu.sync_copy(dst_h.at[pl.ds(k * EPT, EPT)], idx_v)

        ones16 = jnp.ones((16,), _f32)

        def body(j, carry):
            idx16 = idx_v[pl.ds(j * 16, 16)]
            plsc.addupdate_scatter(table, [idx16], ones16)
            return carry

        lax.fori_loop(0, EPT // 16, body, 0)

        for k in range(NS):
            @pl.when((cid == 0) & (sid == k))
            def _(k=k):
                pltpu.sync_copy(table.at[pl.ds(0, N)], out_o.at[k])

            @pl.when((cid == 1) & (sid == k))
            def _(k=k):
                pltpu.sync_copy(table.at[pl.ds(0, N)], out_i.at[k])

    return deg_kernel(src2d, dst2d, zeros1_h)


def _sc_aggregate(hs_list, src2d, dst2d, zeros32_h):
    out_types = [jax.ShapeDtypeStruct((NC, N, C), _f32) for _ in range(NCH)]

    @functools.partial(
        pl.kernel,
        out_type=out_types,
        mesh=_mesh(),
        compiler_params=pltpu.CompilerParams(use_tc_tiling_on_sc=False),
        scratch_types=[
            pltpu.VMEM((ROWS_PER_TILE_AGG, 128), jnp.int32),
            pltpu.VMEM((ROWS_PER_TILE_AGG, 128), jnp.int32),
            pltpu.VMEM((128, C), _f32),
            pltpu.VMEM((128, C), _f32),
            pltpu.VMEM((1024, C), _f32),
            pltpu.VMEM_SHARED((NPAD, C), _f32),
            pltpu.SemaphoreType.DMA,
            pltpu.SemaphoreType.DMA,
        ],
    )
    def agg_kernel(*args):
        hs_refs = args[:NCH]
        src_h, dst_h, zeros_hbm = args[NCH:NCH + 3]
        outs = args[NCH + 3:2 * NCH + 3]
        src_l, dst_l, rows0, rows1, zb, table, sem0, sem1 = args[2 * NCH + 3:]
        cid = lax.axis_index("c")
        sid = lax.axis_index("s")
        pltpu.sync_copy(zeros_hbm, zb)
        rbase = cid * (NS * ROWS_PER_TILE_AGG) + sid * ROWS_PER_TILE_AGG
        pltpu.sync_copy(src_h.at[pl.ds(rbase, ROWS_PER_TILE_AGG)], src_l)
        pltpu.sync_copy(dst_h.at[pl.ds(rbase, ROWS_PER_TILE_AGG)], dst_l)
        base = sid * 3125
        NB = ROWS_PER_TILE_AGG
        for ch in range(NCH):
            for off, sz in _zero_parts(3125, 1024):
                pltpu.sync_copy(zb.at[pl.ds(0, sz)], table.at[pl.ds(base + off, sz)])
            plsc.subcore_barrier()

            pltpu.async_copy(hs_refs[ch].at[src_l.at[0]], rows0, sem0)

            def body(o, carry):
                j = o * 2
                pltpu.make_async_copy(hs_refs[ch].at[src_l.at[0]], rows0, sem0).wait()
                pltpu.async_copy(hs_refs[ch].at[src_l.at[j + 1]], rows1, sem1)
                pltpu.sync_copy(rows0, table.at[dst_l.at[j]], add=True)

                @pl.when(o < NB // 2 - 1)
                def _():
                    pltpu.async_copy(hs_refs[ch].at[src_l.at[j + 2]], rows0, sem0)

                pltpu.make_async_copy(hs_refs[ch].at[src_l.at[0]], rows1, sem1).wait()
                pltpu.sync_copy(rows1, table.at[dst_l.at[j + 1]], add=True)
                return carry

            lax.fori_loop(0, NB // 2, body, 0)
            plsc.subcore_barrier()
            pltpu.sync_copy(
                table.at[pl.ds(base, 3125)], outs[ch].at[cid, pl.ds(base, 3125)]
            )
            plsc.subcore_barrier()

    return agg_kernel(*hs_list, src2d, dst2d, zeros32_h)


def _tc_embed(nodes_feat, W_emb, b_emb2, dgo_p, dgi_p):
    def body(nf, we, be, dgo_r, dgi_r, h_o, ns_o, nd_o, *hs_refs):
        x = jnp.dot(nf[...], we[...], preferred_element_type=_f32,
                    precision=lax.Precision.HIGHEST) + be[...]
        dgo = jnp.sum(dgo_r[...], axis=1, keepdims=True)
        dgi = jnp.sum(dgi_r[...], axis=1, keepdims=True)
        ns = jnp.where(dgo > 0, lax.rsqrt(jnp.maximum(dgo, 1.0)), 0.0)
        nd = jnp.where(dgi > 0, lax.rsqrt(jnp.maximum(dgi, 1.0)), 0.0)
        h_o[...] = x
        ns_o[...] = ns
        nd_o[...] = nd
        hp = x * ns
        hpad = jnp.concatenate([hp, jnp.zeros((BN, FPAD - F), _f32)], axis=1)
        for ch in range(NCH):
            hs_refs[ch][...] = hpad[:, ch * C:(ch + 1) * C]

    out_shapes = (
        [jax.ShapeDtypeStruct((N, F), _f32),
         jax.ShapeDtypeStruct((N, 1), _f32),
         jax.ShapeDtypeStruct((N, 1), _f32)]
        + [jax.ShapeDtypeStruct((NPAD, C), _f32) for _ in range(NCH)]
    )
    out_specs = (
        [pl.BlockSpec((BN, F), lambda i: (i, 0)),
         pl.BlockSpec((BN, 1), lambda i: (i, 0)),
         pl.BlockSpec((BN, 1), lambda i: (i, 0))]
        + [pl.BlockSpec((BN, C), lambda i: (i, 0)) for _ in range(NCH)]
    )
    return pl.pallas_call(
        body,
        grid=(GRID,),
        compiler_params=pltpu.CompilerParams(vmem_limit_bytes=61440000),
        in_specs=[
            pl.BlockSpec((BN, DIN), lambda i: (i, 0)),
            pl.BlockSpec((DIN, F), lambda i: (0, 0)),
            pl.BlockSpec((1, F), lambda i: (0, 0)),
            pl.BlockSpec((BN, NS), lambda i: (i, 0)),
            pl.BlockSpec((BN, NS), lambda i: (i, 0)),
        ],
        out_specs=out_specs,
        out_shape=out_shapes,
    )(nodes_feat, W_emb, b_emb2, dgo_p, dgi_p)


def _tc_linear_stats(agg_list, nd, Wp, b2, snorm):
    def body(*args):
        aggs = args[:NCH]
        nd_r, w_r, b_r, sn_r, z_o, s_o, q_o = args[NCH:]
        i = pl.program_id(0)
        ndv = nd_r[...]
        acc = jnp.zeros((BNB, F), _f32)
        for ch in range(NCH):
            a = (aggs[ch][0] + aggs[ch][1]) * ndv
            acc = acc + jnp.dot(a, w_r[ch * C:(ch + 1) * C, :],
                                preferred_element_type=_f32, precision=lax.Precision.HIGHEST)
        z = (acc + b_r[...]) * sn_r[...]
        z_o[...] = z
        cs = jnp.sum(z, axis=0, keepdims=True)
        cq = jnp.sum(z * z, axis=0, keepdims=True)

        @pl.when(i == 0)
        def _():
            s_o[...] = cs
            q_o[...] = cq

        @pl.when(i > 0)
        def _():
            s_o[...] = s_o[...] + cs
            q_o[...] = q_o[...] + cq

    return pl.pallas_call(
        body,
        grid=(GRIDB,),
        compiler_params=pltpu.CompilerParams(vmem_limit_bytes=61440000),
        in_specs=(
            [pl.BlockSpec((NC, BNB, C), lambda i: (0, i, 0)) for _ in range(NCH)]
            + [pl.BlockSpec((BNB, 1), lambda i: (i, 0)),
               pl.BlockSpec((FPAD, F), lambda i: (0, 0)),
               pl.BlockSpec((1, F), lambda i: (0, 0)),
               pl.BlockSpec((BNB, 1), lambda i: (i, 0))]
        ),
        out_specs=[
            pl.BlockSpec((BNB, F), lambda i: (i, 0)),
            pl.BlockSpec((1, F), lambda i: (0, 0)),
            pl.BlockSpec((1, F), lambda i: (0, 0)),
        ],
        out_shape=[
            jax.ShapeDtypeStruct((N, F), _f32),
            jax.ShapeDtypeStruct((1, F), _f32),
            jax.ShapeDtypeStruct((1, F), _f32),
        ],
    )(*agg_list, nd, Wp, b2, snorm)


def _tc_norm_residual(Z, h, s1, q1, gamma2, beta2, ns):
    def body(z_r, h_r, s_r, q_r, g_r, be_r, ns_r, h2_o, *hs_refs):
        mean = s_r[...] * (1.0 / N)
        var = q_r[...] * (1.0 / N) - mean * mean
        inv = lax.rsqrt(var + 1e-5)
        out = h_r[...] + jnp.maximum((z_r[...] - mean) * inv * g_r[...] + be_r[...], 0.0)
        h2_o[...] = out
        hp = out * ns_r[...]
        hpad = jnp.concatenate([hp, jnp.zeros((BN, FPAD - F), _f32)], axis=1)
        for ch in range(NCH):
            hs_refs[ch][...] = hpad[:, ch * C:(ch + 1) * C]

    out_shapes = (
        [jax.ShapeDtypeStruct((N, F), _f32)]
        + [jax.ShapeDtypeStruct((NPAD, C), _f32) for _ in range(NCH)]
    )
    out_specs = (
        [pl.BlockSpec((BN, F), lambda i: (i, 0))]
        + [pl.BlockSpec((BN, C), lambda i: (i, 0)) for _ in range(NCH)]
    )
    return pl.pallas_call(
        body,
        grid=(GRID,),
        compiler_params=pltpu.CompilerParams(vmem_limit_bytes=61440000),
        in_specs=[
            pl.BlockSpec((BN, F), lambda i: (i, 0)),
            pl.BlockSpec((BN, F), lambda i: (i, 0)),
            pl.BlockSpec((1, F), lambda i: (0, 0)),
            pl.BlockSpec((1, F), lambda i: (0, 0)),
            pl.BlockSpec((1, F), lambda i: (0, 0)),
            pl.BlockSpec((1, F), lambda i: (0, 0)),
            pl.BlockSpec((BN, 1), lambda i: (i, 0)),
        ],
        out_specs=out_specs,
        out_shape=out_shapes,
    )(Z, h, s1, q1, gamma2, beta2, ns)


def _tc_norm_readout(Z, h, s1, q1, gamma2, beta2):
    def body(z_r, h_r, s_r, q_r, g_r, be_r, hg_o):
        i = pl.program_id(0)
        mean = s_r[...] * (1.0 / N)
        var = q_r[...] * (1.0 / N) - mean * mean
        inv = lax.rsqrt(var + 1e-5)
        out = h_r[...] + jnp.maximum((z_r[...] - mean) * inv * g_r[...] + be_r[...], 0.0)
        cs = jnp.sum(out, axis=0, keepdims=True) * (1.0 / N)

        @pl.when(i == 0)
        def _():
            hg_o[...] = cs

        @pl.when(i > 0)
        def _():
            hg_o[...] = hg_o[...] + cs

    return pl.pallas_call(
        body,
        grid=(GRID,),
        compiler_params=pltpu.CompilerParams(vmem_limit_bytes=61440000),
        in_specs=[
            pl.BlockSpec((BN, F), lambda i: (i, 0)),
            pl.BlockSpec((BN, F), lambda i: (i, 0)),
            pl.BlockSpec((1, F), lambda i: (0, 0)),
            pl.BlockSpec((1, F), lambda i: (0, 0)),
            pl.BlockSpec((1, F), lambda i: (0, 0)),
            pl.BlockSpec((1, F), lambda i: (0, 0)),
        ],
        out_specs=pl.BlockSpec((1, F), lambda i: (0, 0)),
        out_shape=jax.ShapeDtypeStruct((1, F), _f32),
    )(Z, h, s1, q1, gamma2, beta2)


def kernel(nodes_feat, edge_index, edges_feat, nodes_num_norm_sqrt,
           edges_num_norm_sqrt, W_emb, b_emb, W1, b1, gamma1, beta1,
           Wo, bo, gammao, betao):
    del edges_feat, edges_num_norm_sqrt
    src = edge_index[0].astype(jnp.int32)
    dst = edge_index[1].astype(jnp.int32)
    fill = jnp.full((EPAD - E,), N, dtype=jnp.int32)
    src_flat = jnp.concatenate([src, fill])
    dst_flat = jnp.concatenate([dst, fill])
    src2d = src_flat.reshape(EPAD // 128, 128)
    dst2d = dst_flat.reshape(EPAD // 128, 128)

    zeros32_h = jnp.zeros((1024, C), _f32)
    zerosN_h = jnp.zeros((NPAD,), _f32)

    b_emb2 = b_emb.reshape(1, F)
    b12 = b1.reshape(1, F)
    bo2 = bo.reshape(1, F)
    gamma12 = gamma1.reshape(1, F)
    beta12 = beta1.reshape(1, F)
    gammao2 = gammao.reshape(1, F)
    betao2 = betao.reshape(1, F)
    W1p = jnp.pad(W1, ((0, FPAD - F), (0, 0)))
    Wop = jnp.pad(Wo, ((0, FPAD - F), (0, 0)))
    snorm = nodes_num_norm_sqrt

    dgo_p, dgi_p = _sc_degrees(src_flat, dst_flat, zerosN_h)
    dgo_p = dgo_p.T
    dgi_p = dgi_p.T

    h, ns, nd, *hs_list = _tc_embed(nodes_feat, W_emb, b_emb2, dgo_p, dgi_p)

    agg1 = _sc_aggregate(hs_list, src2d, dst2d, zeros32_h)
    Z1, s1, q1 = _tc_linear_stats(agg1, nd, W1p, b12, snorm)
    h2, *hs2_list = _tc_norm_residual(Z1, h, s1, q1, gamma12, beta12, ns)

    agg2 = _sc_aggregate(hs2_list, src2d, dst2d, zeros32_h)
    Z2, s2, q2 = _tc_linear_stats(agg2, nd, Wop, bo2, snorm)
    hg = _tc_norm_readout(Z2, h2, s2, q2, gammao2, betao2)
    return hg
